# Initial kernel scaffold; baseline (speedup 1.0000x reference)
import jax
import jax.numpy as jnp
from jax.experimental import pallas as pl


def kernel(sentence_tokens, embedding_table):
    raise NotImplementedError("write your pallas kernel here")


# speedup vs baseline: 1.3572x; 1.3572x over previous
import functools

import jax
import jax.numpy as jnp
from jax import lax
from jax.experimental import pallas as pl
from jax.experimental.pallas import tpu as pltpu
from jax.experimental.pallas import tpu_sc as plsc

VOCAB = 1000000
EMBED_DIM = 32
SEQ_LEN = 200
BATCH = 4096

_INFO = plsc.get_sparse_core_info()
_NC = _INFO.num_cores
_NS = _INFO.num_subcores
_NW = _NC * _NS
_BPW = BATCH // _NW
_L = 16


def _embed_bag_body(tok_hbm, table_hbm, out_hbm, idx_v, rows_v, acc_v, sem):
    wid = lax.axis_index("s") * _NC + lax.axis_index("c")
    wbase = wid * _BPW

    zeros = jnp.zeros((_L,), jnp.float32)
    def zbody(b, _):
        acc_v[b, pl.ds(0, _L)] = zeros
        acc_v[b, pl.ds(_L, _L)] = zeros
        return 0
    lax.fori_loop(0, _BPW, zbody, 0, unroll=8)

    def sbody(s, _):
        pltpu.sync_copy(tok_hbm.at[pl.ds(s * BATCH + wbase, _BPW)], idx_v)
        pltpu.async_copy(table_hbm.at[idx_v], rows_v, sem).wait()

        def abody(b, _):
            r0 = rows_v[b, pl.ds(0, _L)]
            r1 = rows_v[b, pl.ds(_L, _L)]
            plsc.addupdate(acc_v.at[b, pl.ds(0, _L)], r0)
            plsc.addupdate(acc_v.at[b, pl.ds(_L, _L)], r1)
            return 0
        lax.fori_loop(0, _BPW, abody, 0, unroll=8)
        return 0

    lax.fori_loop(0, SEQ_LEN, sbody, 0)

    pltpu.sync_copy(acc_v, out_hbm.at[pl.ds(wbase, _BPW)])


@jax.jit
def kernel(sentence_tokens, embedding_table):
    tok_flat = sentence_tokens.astype(jnp.int32).reshape(-1)
    mesh = plsc.VectorSubcoreMesh(core_axis_name="c", subcore_axis_name="s")
    run = functools.partial(
        pl.kernel,
        out_type=jax.ShapeDtypeStruct((BATCH, EMBED_DIM), jnp.float32),
        mesh=mesh,
        scratch_types=[
            pltpu.VMEM((_BPW,), jnp.int32),
            pltpu.VMEM((_BPW, EMBED_DIM), jnp.float32),
            pltpu.VMEM((_BPW, EMBED_DIM), jnp.float32),
            pltpu.SemaphoreType.DMA,
        ],
        compiler_params=pltpu.CompilerParams(use_tc_tiling_on_sc=False),
    )(_embed_bag_body)
    return run(tok_flat, embedding_table)


# speedup vs baseline: 1.9358x; 1.4263x over previous
import functools

import jax
import jax.numpy as jnp
from jax import lax
from jax.experimental import pallas as pl
from jax.experimental.pallas import tpu as pltpu
from jax.experimental.pallas import tpu_sc as plsc

VOCAB = 1000000
EMBED_DIM = 32
SEQ_LEN = 200
BATCH = 4096

_INFO = plsc.get_sparse_core_info()
_NC = _INFO.num_cores
_NS = _INFO.num_subcores
_NW = _NC * _NS
_BPW = BATCH // _NW
_L = 16
_NBUF = 8
_NGRP = SEQ_LEN // _NBUF


def _embed_bag_body(tok_hbm, table_hbm, out_hbm, idx_all, rows_v, acc_v,
                    *sems):
    wid = lax.axis_index("s") * _NC + lax.axis_index("c")
    wbase = wid * _BPW

    pltpu.sync_copy(tok_hbm.at[:, pl.ds(wbase, _BPW)], idx_all)

    zeros = jnp.zeros((_L,), jnp.float32)

    def zbody(b, _):
        acc_v[b, pl.ds(0, _L)] = zeros
        acc_v[b, pl.ds(_L, _L)] = zeros
        return 0
    lax.fori_loop(0, _BPW, zbody, 0, unroll=8)

    def fire(s, b):
        pltpu.async_copy(table_hbm.at[idx_all.at[s]], rows_v.at[b], sems[b])

    for b in range(_NBUF):
        fire(b, b)

    def grp(g, _):
        for b in range(_NBUF):
            s = g * _NBUF + b
            pltpu.make_async_copy(
                table_hbm.at[idx_all.at[s]], rows_v.at[b], sems[b]).wait()

            def abody(r, _, b=b):
                r0 = rows_v[b, r, pl.ds(0, _L)]
                r1 = rows_v[b, r, pl.ds(_L, _L)]
                plsc.addupdate(acc_v.at[r, pl.ds(0, _L)], r0)
                plsc.addupdate(acc_v.at[r, pl.ds(_L, _L)], r1)
                return 0
            lax.fori_loop(0, _BPW, abody, 0, unroll=8)

            nxt = s + _NBUF

            @pl.when(nxt < SEQ_LEN)
            def _(nxt=nxt, b=b):
                fire(nxt, b)
        return 0
    lax.fori_loop(0, _NGRP, grp, 0)

    pltpu.sync_copy(acc_v, out_hbm.at[pl.ds(wbase, _BPW)])


@jax.jit
def kernel(sentence_tokens, embedding_table):
    tok = sentence_tokens.astype(jnp.int32)
    mesh = plsc.VectorSubcoreMesh(core_axis_name="c", subcore_axis_name="s")
    run = functools.partial(
        pl.kernel,
        out_type=jax.ShapeDtypeStruct((BATCH, EMBED_DIM), jnp.float32),
        mesh=mesh,
        scratch_types=[
            pltpu.VMEM((SEQ_LEN, _BPW), jnp.int32),
            pltpu.VMEM((_NBUF, _BPW, EMBED_DIM), jnp.float32),
            pltpu.VMEM((_BPW, EMBED_DIM), jnp.float32),
        ] + [pltpu.SemaphoreType.DMA] * _NBUF,
        compiler_params=pltpu.CompilerParams(use_tc_tiling_on_sc=False),
    )(_embed_bag_body)
    return run(tok, embedding_table)
